# baseline (speedup 1.0000x reference)
import functools
import numpy as np

import jax
import jax.numpy as jnp
from jax.experimental import pallas as pl
from jax.experimental.pallas import tpu as pltpu


def _shift(a, s, hw):
    return a if s == 0 else pltpu.roll(a, (-s) % hw, axis=a.ndim - 1)


def _fused_kernel(x_ref, w_ref, b_ref, wc_ref, t_ref, c_ref, o_ref, col_ref,
                  *, W, O, K):
    m = x_ref[...]
    C, HW = m.shape

    maskH = c_ref[0:1, :]
    maskV = c_ref[1:2, :]
    maskD = c_ref[2:3, :]
    selR = c_ref[3:4, :]
    selD = c_ref[4:5, :]
    tbHS = c_ref[5:6, :]
    tbVS = c_ref[6:7, :]
    tbDS = c_ref[7:8, :]
    tbHV = c_ref[8:9, :]
    tbHD = c_ref[9:10, :]
    tbVD = c_ref[10:11, :]
    mxm = c_ref[11:12, :]
    mxp = c_ref[12:13, :]
    mym = c_ref[13:14, :]
    myp = c_ref[14:15, :]

    rp1 = _shift(m, 1, HW)
    rm1 = _shift(m, -1, HW)
    vH = rm1 + selR * (rp1 - rm1)
    rpW = _shift(m, W, HW)
    rmW = _shift(m, -W, HW)
    vV = rmW + selD * (rpW - rmW)
    vp1 = _shift(vV, 1, HW)
    vm1 = _shift(vV, -1, HW)
    vD = vm1 + selR * (vp1 - vm1)

    ss = jnp.sum(m * m, axis=0, keepdims=True)
    sp1 = _shift(ss, 1, HW)
    sm1 = _shift(ss, -1, HW)
    ssH = (sm1 + selR * (sp1 - sm1)) * maskH
    spW = _shift(ss, W, HW)
    smW = _shift(ss, -W, HW)
    ssVu = smW + selD * (spW - smW)
    ssV = ssVu * maskV
    svp = _shift(ssVu, 1, HW)
    svm = _shift(ssVu, -1, HW)
    ssD = (svm + selR * (svp - svm)) * maskD
    pdH = 2.0 * (jnp.sum(m * vH, axis=0, keepdims=True) * maskH) - ss - ssH
    pdV = 2.0 * (jnp.sum(m * vV, axis=0, keepdims=True) * maskV) - ss - ssV
    pdD = 2.0 * (jnp.sum(m * vD, axis=0, keepdims=True) * maskD) - ss - ssD
    pdS = jnp.zeros_like(ss)

    def bet(pa, pq, tb):
        return ((pa > pq) | ((pa == pq) & (tb > 0.5))).astype(jnp.float32)

    one = jnp.float32(1.0)
    bHS = bet(pdH, pdS, tbHS)
    bVS = bet(pdV, pdS, tbVS)
    bDS = bet(pdD, pdS, tbDS)
    bHV = bet(pdH, pdV, tbHV)
    bHD = bet(pdH, pdD, tbHD)
    bVD = bet(pdV, pdD, tbVD)
    rankS = bHS + bVS + bDS
    rankH = (one - bHS) + (one - bHV) + (one - bHD)
    rankV = (one - bVS) + bHV + (one - bVD)
    rankD = (one - bDS) + bHD + bVD

    mulS = jax.lax.rsqrt(jnp.maximum(ss, 1e-24))
    mulH = jax.lax.rsqrt(jnp.maximum(ssH, 1e-24)) * maskH
    mulV = jax.lax.rsqrt(jnp.maximum(ssV, 1e-24)) * maskV
    mulD = jax.lax.rsqrt(jnp.maximum(ssD, 1e-24)) * maskD

    def coef(rank, mul, i):
        c = w_ref[i * K + 0] * (rank == 0.0).astype(jnp.float32)
        for j in range(1, K):
            c = c + w_ref[i * K + j] * (rank == jnp.float32(j)).astype(jnp.float32)
        return c * mul

    gout = None
    for i in range(K):
        pre = (coef(rankS, mulS, i) * m + coef(rankH, mulH, i) * vH
               + coef(rankV, mulV, i) * vV + coef(rankD, mulD, i) * vD)
        yi = jnp.maximum(pre + b_ref[i], 0.0)
        gout = yi if gout is None else jnp.maximum(gout, yi)

    gbf = gout.astype(jnp.bfloat16)
    col_ref[0:C, :] = _shift(gbf, -1, HW) * mxm.astype(jnp.bfloat16)
    col_ref[C:2 * C, :] = gbf
    col_ref[2 * C:3 * C, :] = _shift(gbf, 1, HW) * mxp.astype(jnp.bfloat16)

    p = jnp.dot(wc_ref[...], col_ref[...], preferred_element_type=jnp.float32)
    z = (p[O:2 * O] + _shift(p[0:O], -W, HW) * mym
         + _shift(p[2 * O:3 * O], W, HW) * myp)
    z = jnp.maximum(z + t_ref[...], 0.0)

    zx = jnp.maximum(z, jnp.maximum(_shift(z, 1, HW) * mxp,
                                    _shift(z, -1, HW) * mxm))
    out = jnp.maximum(zx, jnp.maximum(_shift(zx, W, HW) * myp,
                                      _shift(zx, -W, HW) * mym))
    o_ref[...] = out


def _build_consts(H, W):
    HW = H * W
    ys, xs = np.arange(HW) // W, np.arange(HW) % W
    xodd = (xs % 2 == 1)
    yodd = (ys % 2 == 1)
    c = np.zeros((15, HW), np.float32)
    c[0] = np.where(xodd, xs + 1 < W, xs - 1 >= 0)
    c[1] = np.where(yodd, ys + 1 < H, ys - 1 >= 0)
    c[2] = c[0] * c[1]
    c[3] = xodd
    c[4] = yodd
    c[5] = ~xodd
    c[6] = ~yodd
    c[7] = ~yodd
    c[8] = yodd
    c[9] = yodd
    c[10] = xodd
    c[11] = xs - 1 >= 0
    c[12] = xs + 1 < W
    c[13] = ys - 1 >= 0
    c[14] = ys + 1 < H
    return jnp.asarray(c)


def kernel(x, w_fc, b_fc, w_conv, b_conv, bn_gamma, bn_beta, bn_mean, bn_var):
    B, C, H, W = x.shape
    O = w_conv.shape[0]
    HW = H * W
    K = 4
    eps = 1e-5

    s = bn_gamma / jnp.sqrt(bn_var + eps)
    t = ((b_conv - bn_mean) * s + bn_beta).reshape(O, 1)
    w_eff = ((w_conv * s[:, None, None, None])
             .transpose(2, 0, 3, 1).reshape(3 * O, 3 * C).astype(jnp.bfloat16))
    consts = _build_consts(H, W)

    out = pl.pallas_call(
        functools.partial(_fused_kernel, W=W, O=O, K=K),
        out_shape=jax.ShapeDtypeStruct((B * O, HW), jnp.float32),
        grid=(B,),
        in_specs=[
            pl.BlockSpec((C, HW), lambda b: (b, 0)),
            pl.BlockSpec(memory_space=pltpu.MemorySpace.SMEM),
            pl.BlockSpec(memory_space=pltpu.MemorySpace.SMEM),
            pl.BlockSpec((3 * O, 3 * C), lambda b: (0, 0)),
            pl.BlockSpec((O, 1), lambda b: (0, 0)),
            pl.BlockSpec((15, HW), lambda b: (0, 0)),
        ],
        out_specs=pl.BlockSpec((O, HW), lambda b: (b, 0)),
        scratch_shapes=[pltpu.VMEM((3 * C, HW), jnp.bfloat16)],
        compiler_params=pltpu.CompilerParams(dimension_semantics=("parallel",)),
    )(x.reshape(B * C, HW), w_fc.reshape(-1).astype(jnp.float32),
      b_fc.astype(jnp.float32), w_eff, t, consts)
    return out.reshape(B, O, H, W)


# speedup vs baseline: 1.0091x; 1.0091x over previous
import functools
import numpy as np

import jax
import jax.numpy as jnp
from jax.experimental import pallas as pl
from jax.experimental.pallas import tpu as pltpu


def _shift(a, s, hw):
    return a if s == 0 else pltpu.roll(a, (-s) % hw, axis=a.ndim - 1)


def _fused_kernel(x_ref, w_ref, b_ref, wc_ref, t_ref, c_ref, o_ref, col_ref,
                  *, W, O, K, IMGS):
    C = x_ref.shape[0] // IMGS
    HW = x_ref.shape[1]

    maskH = c_ref[0:1, :]
    maskV = c_ref[1:2, :]
    maskD = c_ref[2:3, :]
    selR = c_ref[3:4, :]
    selD = c_ref[4:5, :]
    tbHS = c_ref[5:6, :]
    tbVS = c_ref[6:7, :]
    tbDS = c_ref[7:8, :]
    tbHV = c_ref[8:9, :]
    tbHD = c_ref[9:10, :]
    tbVD = c_ref[10:11, :]
    mxm = c_ref[11:12, :]
    mxp = c_ref[12:13, :]
    mym = c_ref[13:14, :]
    myp = c_ref[14:15, :]

    for img in range(IMGS):
      m = x_ref[img * C:(img + 1) * C, :]
      if True:
        rp1 = _shift(m, 1, HW)
        rm1 = _shift(m, -1, HW)
        vH = rm1 + selR * (rp1 - rm1)
        rpW = _shift(m, W, HW)
        rmW = _shift(m, -W, HW)
        vV = rmW + selD * (rpW - rmW)
        vp1 = _shift(vV, 1, HW)
        vm1 = _shift(vV, -1, HW)
        vD = vm1 + selR * (vp1 - vm1)

        ss = jnp.sum(m * m, axis=0, keepdims=True)
        sp1 = _shift(ss, 1, HW)
        sm1 = _shift(ss, -1, HW)
        ssH = (sm1 + selR * (sp1 - sm1)) * maskH
        spW = _shift(ss, W, HW)
        smW = _shift(ss, -W, HW)
        ssVu = smW + selD * (spW - smW)
        ssV = ssVu * maskV
        svp = _shift(ssVu, 1, HW)
        svm = _shift(ssVu, -1, HW)
        ssD = (svm + selR * (svp - svm)) * maskD
        pdH = 2.0 * (jnp.sum(m * vH, axis=0, keepdims=True) * maskH) - ss - ssH
        pdV = 2.0 * (jnp.sum(m * vV, axis=0, keepdims=True) * maskV) - ss - ssV
        pdD = 2.0 * (jnp.sum(m * vD, axis=0, keepdims=True) * maskD) - ss - ssD
        pdS = jnp.zeros_like(ss)

        def bet(pa, pq, tb):
            return ((pa > pq) | ((pa == pq) & (tb > 0.5))).astype(jnp.float32)

        one = jnp.float32(1.0)
        bHS = bet(pdH, pdS, tbHS)
        bVS = bet(pdV, pdS, tbVS)
        bDS = bet(pdD, pdS, tbDS)
        bHV = bet(pdH, pdV, tbHV)
        bHD = bet(pdH, pdD, tbHD)
        bVD = bet(pdV, pdD, tbVD)
        rankS = bHS + bVS + bDS
        rankH = (one - bHS) + (one - bHV) + (one - bHD)
        rankV = (one - bVS) + bHV + (one - bVD)
        rankD = (one - bDS) + bHD + bVD

        mulS = jax.lax.rsqrt(jnp.maximum(ss, 1e-24))
        mulH = jax.lax.rsqrt(jnp.maximum(ssH, 1e-24)) * maskH
        mulV = jax.lax.rsqrt(jnp.maximum(ssV, 1e-24)) * maskV
        mulD = jax.lax.rsqrt(jnp.maximum(ssD, 1e-24)) * maskD

        def coef(rank, mul, i):
            c = w_ref[i * K + 0] * (rank == 0.0).astype(jnp.float32)
            for j in range(1, K):
                c = c + w_ref[i * K + j] * (rank == jnp.float32(j)).astype(jnp.float32)
            return c * mul

        gout = None
        for i in range(K):
            pre = (coef(rankS, mulS, i) * m + coef(rankH, mulH, i) * vH
                   + coef(rankV, mulV, i) * vV + coef(rankD, mulD, i) * vD)
            yi = jnp.maximum(pre + b_ref[i], 0.0)
            gout = yi if gout is None else jnp.maximum(gout, yi)

        gbf = gout.astype(jnp.bfloat16)
        col_ref[img * 3 * C:img * 3 * C + C, :] = _shift(gbf, -1, HW) * mxm.astype(jnp.bfloat16)
        col_ref[img * 3 * C + C:img * 3 * C + 2 * C, :] = gbf
        col_ref[img * 3 * C + 2 * C:img * 3 * C + 3 * C, :] = _shift(gbf, 1, HW) * mxp.astype(jnp.bfloat16)

        p = jnp.dot(wc_ref[...], col_ref[img * 3 * C:(img + 1) * 3 * C, :], preferred_element_type=jnp.float32)
        z = (p[O:2 * O] + _shift(p[0:O], -W, HW) * mym
             + _shift(p[2 * O:3 * O], W, HW) * myp)
        z = jnp.maximum(z + t_ref[...], 0.0)

        zx = jnp.maximum(z, jnp.maximum(_shift(z, 1, HW) * mxp,
                                        _shift(z, -1, HW) * mxm))
        out = jnp.maximum(zx, jnp.maximum(_shift(zx, W, HW) * myp,
                                          _shift(zx, -W, HW) * mym))
        o_ref[img * O:(img + 1) * O, :] = out


def _build_consts(H, W):
    HW = H * W
    ys, xs = np.arange(HW) // W, np.arange(HW) % W
    xodd = (xs % 2 == 1)
    yodd = (ys % 2 == 1)
    c = np.zeros((15, HW), np.float32)
    c[0] = np.where(xodd, xs + 1 < W, xs - 1 >= 0)
    c[1] = np.where(yodd, ys + 1 < H, ys - 1 >= 0)
    c[2] = c[0] * c[1]
    c[3] = xodd
    c[4] = yodd
    c[5] = ~xodd
    c[6] = ~yodd
    c[7] = ~yodd
    c[8] = yodd
    c[9] = yodd
    c[10] = xodd
    c[11] = xs - 1 >= 0
    c[12] = xs + 1 < W
    c[13] = ys - 1 >= 0
    c[14] = ys + 1 < H
    return jnp.asarray(c)


def kernel(x, w_fc, b_fc, w_conv, b_conv, bn_gamma, bn_beta, bn_mean, bn_var):
    B, C, H, W = x.shape
    O = w_conv.shape[0]
    HW = H * W
    K = 4
    eps = 1e-5

    s = bn_gamma / jnp.sqrt(bn_var + eps)
    t = ((b_conv - bn_mean) * s + bn_beta).reshape(O, 1)
    w_eff = ((w_conv * s[:, None, None, None])
             .transpose(2, 0, 3, 1).reshape(3 * O, 3 * C).astype(jnp.bfloat16))
    consts = _build_consts(H, W)

    IMGS = 2 if B % 2 == 0 else 1
    out = pl.pallas_call(
        functools.partial(_fused_kernel, W=W, O=O, K=K, IMGS=IMGS),
        out_shape=jax.ShapeDtypeStruct((B * O, HW), jnp.float32),
        grid=(B // IMGS,),
        in_specs=[
            pl.BlockSpec((IMGS * C, HW), lambda b: (b, 0)),
            pl.BlockSpec(memory_space=pltpu.MemorySpace.SMEM),
            pl.BlockSpec(memory_space=pltpu.MemorySpace.SMEM),
            pl.BlockSpec((3 * O, 3 * C), lambda b: (0, 0)),
            pl.BlockSpec((O, 1), lambda b: (0, 0)),
            pl.BlockSpec((15, HW), lambda b: (0, 0)),
        ],
        out_specs=pl.BlockSpec((IMGS * O, HW), lambda b: (b, 0)),
        scratch_shapes=[pltpu.VMEM((IMGS * 3 * C, HW), jnp.bfloat16)],
        compiler_params=pltpu.CompilerParams(
            dimension_semantics=("parallel",)),
    )(x.reshape(B * C, HW), w_fc.reshape(-1).astype(jnp.float32),
      b_fc.astype(jnp.float32), w_eff, t, consts)
    return out.reshape(B, O, H, W)


# speedup vs baseline: 1.3351x; 1.3230x over previous
import functools
import numpy as np

import jax
import jax.numpy as jnp
from jax.experimental import pallas as pl
from jax.experimental.pallas import tpu as pltpu


def _shift(a, s, hw):
    return a if s == 0 else pltpu.roll(a, (-s) % hw, axis=a.ndim - 1)


def _fused_kernel(x_ref, w_ref, b_ref, wc_ref, t_ref, c_ref, o_ref, col_ref,
                  *, W, K):
    m = x_ref[0]
    C, HW = m.shape
    one = jnp.float32(1.0)

    maskH = c_ref[0:1, :]
    maskV = c_ref[1:2, :]
    maskD = c_ref[2:3, :]
    selR = c_ref[3:4, :]
    selD = c_ref[4:5, :]
    tbHS = c_ref[5:6, :]
    tbVS = c_ref[6:7, :]
    tbDS = c_ref[7:8, :]
    tbHV = c_ref[8:9, :]
    tbHD = c_ref[9:10, :]
    tbVD = c_ref[10:11, :]

    rp1, rm1 = _shift(m, 1, HW), _shift(m, -1, HW)
    vH = (selR * rp1 + (one - selR) * rm1) * maskH
    rpW, rmW = _shift(m, W, HW), _shift(m, -W, HW)
    vV = (selD * rpW + (one - selD) * rmW) * maskV
    rpp = _shift(m, W + 1, HW)
    rpm = _shift(m, W - 1, HW)
    rmp = _shift(m, -(W - 1), HW)
    rmm = _shift(m, -(W + 1), HW)
    sRD = selR * selD
    vD = (sRD * rpp + (selD - sRD) * rpm + (selR - sRD) * rmp
          + (one - selD - selR + sRD) * rmm) * maskD

    ss = jnp.sum(m * m, axis=0, keepdims=True)
    ssH = (selR * _shift(ss, 1, HW) + (one - selR) * _shift(ss, -1, HW)) * maskH
    ssV = (selD * _shift(ss, W, HW) + (one - selD) * _shift(ss, -W, HW)) * maskV
    ssD = (sRD * _shift(ss, W + 1, HW) + (selD - sRD) * _shift(ss, W - 1, HW)
           + (selR - sRD) * _shift(ss, -(W - 1), HW)
           + (one - selD - selR + sRD) * _shift(ss, -(W + 1), HW)) * maskD
    pdH = 2.0 * jnp.sum(m * vH, axis=0, keepdims=True) - ss - ssH
    pdV = 2.0 * jnp.sum(m * vV, axis=0, keepdims=True) - ss - ssV
    pdD = 2.0 * jnp.sum(m * vD, axis=0, keepdims=True) - ss - ssD
    pdS = jnp.zeros_like(ss)

    def bet(pa, pq, tb):
        return ((pa > pq) | ((pa == pq) & (tb > 0.5))).astype(jnp.float32)

    bHS = bet(pdH, pdS, tbHS)
    bVS = bet(pdV, pdS, tbVS)
    bDS = bet(pdD, pdS, tbDS)
    bHV = bet(pdH, pdV, tbHV)
    bHD = bet(pdH, pdD, tbHD)
    bVD = bet(pdV, pdD, tbVD)
    rankS = bHS + bVS + bDS
    rankH = (one - bHS) + (one - bHV) + (one - bHD)
    rankV = (one - bVS) + bHV + (one - bVD)
    rankD = (one - bDS) + bHD + bVD

    invS = jax.lax.rsqrt(jnp.maximum(ss, 1e-24))
    invH = jax.lax.rsqrt(jnp.maximum(ssH, 1e-24))
    invV = jax.lax.rsqrt(jnp.maximum(ssV, 1e-24))
    invD = jax.lax.rsqrt(jnp.maximum(ssD, 1e-24))

    def coef(rank, inv, i):
        c = w_ref[i * K + 0] * (rank == 0.0).astype(jnp.float32)
        for j in range(1, K):
            c = c + w_ref[i * K + j] * (rank == jnp.float32(j)).astype(jnp.float32)
        return c * inv

    gout = None
    for i in range(K):
        pre = (coef(rankS, invS, i) * m + coef(rankH, invH, i) * vH
               + coef(rankV, invV, i) * vV + coef(rankD, invD, i) * vD)
        yi = jnp.maximum(pre + b_ref[i], 0.0)
        gout = yi if gout is None else jnp.maximum(gout, yi)

    for dy in (-1, 0, 1):
        for dx in (-1, 0, 1):
            tap = (dy + 1) * 3 + (dx + 1)
            shifted = _shift(gout, dy * W + dx, HW)
            if not (dy == 0 and dx == 0):
                shifted = shifted * c_ref[11 + tap:12 + tap, :]
            col_ref[tap * C:(tap + 1) * C, :] = shifted.astype(jnp.bfloat16)

    z = jnp.dot(wc_ref[...], col_ref[...], preferred_element_type=jnp.float32)
    z = jnp.maximum(z + t_ref[...], 0.0)

    out = z
    for dy in (-1, 0, 1):
        for dx in (-1, 0, 1):
            if dy == 0 and dx == 0:
                continue
            tap = (dy + 1) * 3 + (dx + 1)
            shifted = _shift(z, dy * W + dx, HW)
            out = jnp.maximum(out, shifted * c_ref[11 + tap:12 + tap, :])
    o_ref[0] = out


def _build_consts(H, W):
    HW = H * W
    ys, xs = np.arange(HW) // W, np.arange(HW) % W
    xodd = (xs % 2 == 1)
    yodd = (ys % 2 == 1)
    c = np.zeros((20, HW), np.float32)
    c[0] = np.where(xodd, xs + 1 < W, xs - 1 >= 0)
    c[1] = np.where(yodd, ys + 1 < H, ys - 1 >= 0)
    c[2] = c[0] * c[1]
    c[3] = xodd
    c[4] = yodd
    c[5] = ~xodd
    c[6] = ~yodd
    c[7] = ~yodd
    c[8] = yodd
    c[9] = yodd
    c[10] = xodd
    for dy in (-1, 0, 1):
        for dx in (-1, 0, 1):
            ok = (ys + dy >= 0) & (ys + dy < H) & (xs + dx >= 0) & (xs + dx < W)
            c[11 + (dy + 1) * 3 + (dx + 1)] = ok
    return jnp.asarray(c)


def kernel(x, w_fc, b_fc, w_conv, b_conv, bn_gamma, bn_beta, bn_mean, bn_var):
    B, C, H, W = x.shape
    O = w_conv.shape[0]
    HW = H * W
    K = 4
    eps = 1e-5

    s = bn_gamma / jnp.sqrt(bn_var + eps)
    t = ((b_conv - bn_mean) * s + bn_beta).reshape(O, 1)
    w_eff = ((w_conv * s[:, None, None, None])
             .transpose(0, 2, 3, 1).reshape(O, 9 * C).astype(jnp.bfloat16))
    consts = _build_consts(H, W)

    out = pl.pallas_call(
        functools.partial(_fused_kernel, W=W, K=K),
        out_shape=jax.ShapeDtypeStruct((B, O, HW), jnp.float32),
        grid=(B,),
        in_specs=[
            pl.BlockSpec((1, C, HW), lambda b: (b, 0, 0)),
            pl.BlockSpec(memory_space=pltpu.MemorySpace.SMEM),
            pl.BlockSpec(memory_space=pltpu.MemorySpace.SMEM),
            pl.BlockSpec((O, 9 * C), lambda b: (0, 0)),
            pl.BlockSpec((O, 1), lambda b: (0, 0)),
            pl.BlockSpec((20, HW), lambda b: (0, 0)),
        ],
        out_specs=pl.BlockSpec((1, O, HW), lambda b: (b, 0, 0)),
        scratch_shapes=[pltpu.VMEM((9 * C, HW), jnp.bfloat16)],
        compiler_params=pltpu.CompilerParams(dimension_semantics=("parallel",)),
    )(x.reshape(B, C, HW), w_fc.reshape(-1).astype(jnp.float32),
      b_fc.astype(jnp.float32), w_eff, t, consts)
    return out.reshape(B, O, H, W)


# speedup vs baseline: 1.5269x; 1.1437x over previous
import functools
import numpy as np

import jax
import jax.numpy as jnp
from jax.experimental import pallas as pl
from jax.experimental.pallas import tpu as pltpu


def _shift(a, s, hw):
    return a if s == 0 else pltpu.roll(a, (-s) % hw, axis=a.ndim - 1)


def _fused_kernel(x_ref, w_ref, b_ref, wc_ref, t_ref, c_ref, o_ref, col_ref,
                  *, W, K):
    m = x_ref[0]
    C, HW = m.shape
    one = jnp.float32(1.0)

    maskH = c_ref[0:1, :]
    maskV = c_ref[1:2, :]
    maskD = c_ref[2:3, :]
    selR = c_ref[3:4, :]
    selD = c_ref[4:5, :]
    tbHS = c_ref[5:6, :]
    tbVS = c_ref[6:7, :]
    tbDS = c_ref[7:8, :]
    tbHV = c_ref[8:9, :]
    tbHD = c_ref[9:10, :]
    tbVD = c_ref[10:11, :]

    rp1, rm1 = _shift(m, 1, HW), _shift(m, -1, HW)
    vH = (selR * rp1 + (one - selR) * rm1) * maskH
    rpW, rmW = _shift(m, W, HW), _shift(m, -W, HW)
    vV = (selD * rpW + (one - selD) * rmW) * maskV
    rpp = _shift(m, W + 1, HW)
    rpm = _shift(m, W - 1, HW)
    rmp = _shift(m, -(W - 1), HW)
    rmm = _shift(m, -(W + 1), HW)
    sRD = selR * selD
    vD = (sRD * rpp + (selD - sRD) * rpm + (selR - sRD) * rmp
          + (one - selD - selR + sRD) * rmm) * maskD

    ss = jnp.sum(m * m, axis=0, keepdims=True)
    ssH = (selR * _shift(ss, 1, HW) + (one - selR) * _shift(ss, -1, HW)) * maskH
    ssV = (selD * _shift(ss, W, HW) + (one - selD) * _shift(ss, -W, HW)) * maskV
    ssD = (sRD * _shift(ss, W + 1, HW) + (selD - sRD) * _shift(ss, W - 1, HW)
           + (selR - sRD) * _shift(ss, -(W - 1), HW)
           + (one - selD - selR + sRD) * _shift(ss, -(W + 1), HW)) * maskD
    pdH = 2.0 * jnp.sum(m * vH, axis=0, keepdims=True) - ss - ssH
    pdV = 2.0 * jnp.sum(m * vV, axis=0, keepdims=True) - ss - ssV
    pdD = 2.0 * jnp.sum(m * vD, axis=0, keepdims=True) - ss - ssD
    pdS = jnp.zeros_like(ss)

    def bet(pa, pq, tb):
        return ((pa > pq) | ((pa == pq) & (tb > 0.5))).astype(jnp.float32)

    bHS = bet(pdH, pdS, tbHS)
    bVS = bet(pdV, pdS, tbVS)
    bDS = bet(pdD, pdS, tbDS)
    bHV = bet(pdH, pdV, tbHV)
    bHD = bet(pdH, pdD, tbHD)
    bVD = bet(pdV, pdD, tbVD)
    rankS = bHS + bVS + bDS
    rankH = (one - bHS) + (one - bHV) + (one - bHD)
    rankV = (one - bVS) + bHV + (one - bVD)
    rankD = (one - bDS) + bHD + bVD

    invS = jax.lax.rsqrt(jnp.maximum(ss, 1e-24))
    invH = jax.lax.rsqrt(jnp.maximum(ssH, 1e-24))
    invV = jax.lax.rsqrt(jnp.maximum(ssV, 1e-24))
    invD = jax.lax.rsqrt(jnp.maximum(ssD, 1e-24))

    def coef(rank, inv, i):
        c = w_ref[i * K + 0] * (rank == 0.0).astype(jnp.float32)
        for j in range(1, K):
            c = c + w_ref[i * K + j] * (rank == jnp.float32(j)).astype(jnp.float32)
        return c * inv

    gout = None
    for i in range(K):
        pre = (coef(rankS, invS, i) * m + coef(rankH, invH, i) * vH
               + coef(rankV, invV, i) * vV + coef(rankD, invD, i) * vD)
        yi = jnp.maximum(pre + b_ref[i], 0.0)
        gout = yi if gout is None else jnp.maximum(gout, yi)

    gbf = gout.astype(jnp.bfloat16)
    for dy in (-1, 0, 1):
        for dx in (-1, 0, 1):
            tap = (dy + 1) * 3 + (dx + 1)
            shifted = _shift(gbf, dy * W + dx, HW)
            if not (dy == 0 and dx == 0):
                shifted = shifted * c_ref[11 + tap:12 + tap, :].astype(jnp.bfloat16)
            col_ref[tap * C:(tap + 1) * C, :] = shifted

    z = jnp.dot(wc_ref[...], col_ref[...], preferred_element_type=jnp.float32)
    z = jnp.maximum(z + t_ref[...], 0.0)

    mxm = c_ref[14:15, :]
    mxp = c_ref[16:17, :]
    mym = c_ref[12:13, :]
    myp = c_ref[18:19, :]
    zx = jnp.maximum(z, jnp.maximum(_shift(z, 1, HW) * mxp,
                                    _shift(z, -1, HW) * mxm))
    out = jnp.maximum(zx, jnp.maximum(_shift(zx, W, HW) * myp,
                                      _shift(zx, -W, HW) * mym))
    o_ref[0] = out


def _build_consts(H, W):
    HW = H * W
    ys, xs = np.arange(HW) // W, np.arange(HW) % W
    xodd = (xs % 2 == 1)
    yodd = (ys % 2 == 1)
    c = np.zeros((20, HW), np.float32)
    c[0] = np.where(xodd, xs + 1 < W, xs - 1 >= 0)
    c[1] = np.where(yodd, ys + 1 < H, ys - 1 >= 0)
    c[2] = c[0] * c[1]
    c[3] = xodd
    c[4] = yodd
    c[5] = ~xodd
    c[6] = ~yodd
    c[7] = ~yodd
    c[8] = yodd
    c[9] = yodd
    c[10] = xodd
    for dy in (-1, 0, 1):
        for dx in (-1, 0, 1):
            ok = (ys + dy >= 0) & (ys + dy < H) & (xs + dx >= 0) & (xs + dx < W)
            c[11 + (dy + 1) * 3 + (dx + 1)] = ok
    return jnp.asarray(c)


def kernel(x, w_fc, b_fc, w_conv, b_conv, bn_gamma, bn_beta, bn_mean, bn_var):
    B, C, H, W = x.shape
    O = w_conv.shape[0]
    HW = H * W
    K = 4
    eps = 1e-5

    s = bn_gamma / jnp.sqrt(bn_var + eps)
    t = ((b_conv - bn_mean) * s + bn_beta).reshape(O, 1)
    w_eff = ((w_conv * s[:, None, None, None])
             .transpose(0, 2, 3, 1).reshape(O, 9 * C).astype(jnp.bfloat16))
    consts = _build_consts(H, W)

    out = pl.pallas_call(
        functools.partial(_fused_kernel, W=W, K=K),
        out_shape=jax.ShapeDtypeStruct((B, O, HW), jnp.float32),
        grid=(B,),
        in_specs=[
            pl.BlockSpec((1, C, HW), lambda b: (b, 0, 0)),
            pl.BlockSpec(memory_space=pltpu.MemorySpace.SMEM),
            pl.BlockSpec(memory_space=pltpu.MemorySpace.SMEM),
            pl.BlockSpec((O, 9 * C), lambda b: (0, 0)),
            pl.BlockSpec((O, 1), lambda b: (0, 0)),
            pl.BlockSpec((20, HW), lambda b: (0, 0)),
        ],
        out_specs=pl.BlockSpec((1, O, HW), lambda b: (b, 0, 0)),
        scratch_shapes=[pltpu.VMEM((9 * C, HW), jnp.bfloat16)],
        compiler_params=pltpu.CompilerParams(dimension_semantics=("parallel",)),
    )(x.reshape(B, C, HW), w_fc.reshape(-1).astype(jnp.float32),
      b_fc.astype(jnp.float32), w_eff, t, consts)
    return out.reshape(B, O, H, W)
